# baseline (speedup 1.0000x reference)
import functools
import numpy as np
import jax
import jax.numpy as jnp
from jax import lax
from jax.experimental import pallas as pl
from jax.experimental.pallas import tpu as pltpu
from jax.experimental.pallas import tpu_sc as plsc

N_NODES = 10000
N_GRAPHS = 512
NUM_ITEMS = 100000
D = 128
NUM_TREES = 16
TREE_DEPTH = 5
NUM_LEAVES = 32
MAX_DEG = 8
EPS = 1e-5
PI_ROWS = NUM_TREES * NUM_LEAVES


def _bn(x, g, b):
    m = x.mean(axis=0)
    v = x.var(axis=0)
    return g * (x - m) / jnp.sqrt(v + EPS) + b


def _prelu(x, a):
    return jnp.where(x > 0, x, a * x)


def _seg_softmax(e, seg, n):
    mx = jax.ops.segment_max(e, seg, num_segments=n)
    mx = jnp.where(jnp.isfinite(mx), mx, 0.0)
    ex = jnp.exp(e - mx[seg])
    s = jax.ops.segment_sum(ex, seg, num_segments=n)
    return ex / jnp.maximum(s[seg], 1e-12)


NW = 32


GATHER_NB = 4


def _sc_gather_rows(table, idx2, n_rows):
    nch = n_rows // 128
    assert nch % (NW * GATHER_NB) == 0
    nc_w = nch // NW
    mesh = plsc.VectorSubcoreMesh(core_axis_name="c", subcore_axis_name="s")

    @functools.partial(
        pl.kernel, mesh=mesh,
        out_type=jax.ShapeDtypeStruct((n_rows, 128), jnp.float32),
        scratch_types=(
            [pltpu.VMEM((128,), jnp.int32) for _ in range(GATHER_NB)]
            + [pltpu.VMEM((128, 128), jnp.float32) for _ in range(GATHER_NB)]
            + [pltpu.SemaphoreType.DMA]
        ),
    )
    def gk(table_hbm, idx_hbm, out_hbm, *rest):
        idxb = rest[:GATHER_NB]
        bufs = rest[GATHER_NB:2 * GATHER_NB]
        sem = rest[2 * GATHER_NB]
        wid = lax.axis_index("s") * 2 + lax.axis_index("c")
        c0 = wid * nc_w
        for g in range(0, nc_w, GATHER_NB):
            for b in range(GATHER_NB):
                pltpu.sync_copy(
                    idx_hbm.at[pl.ds((c0 + g + b) * 128, 128)], idxb[b])
            descs = [
                pltpu.async_copy(table_hbm.at[idxb[b]], bufs[b], sem)
                for b in range(GATHER_NB)
            ]
            for b in range(GATHER_NB):
                descs[b].wait()
                pltpu.sync_copy(
                    bufs[b], out_hbm.at[pl.ds((c0 + g + b) * 128, 128)])

    return gk(table, idx2)


def _pad_idx(idx, mult):
    n = idx.shape[0]
    npad = ((n + mult - 1) // mult) * mult
    return jnp.pad(idx, (0, npad - n)), npad


GRU_NT = 2000


def _gru_body(x_ref, mask_ref, wih_ref, whh_ref, bih_ref, bhh_ref, h_ref):
    k = pl.program_id(1)

    @pl.when(k == 0)
    def _():
        h_ref[...] = jnp.zeros_like(h_ref)

    x = x_ref[0].astype(jnp.float32)
    h = h_ref[...]
    gi = lax.dot_general(x, wih_ref[...], (((1,), (1,)), ((), ())),
                         preferred_element_type=jnp.float32) + bih_ref[...]
    gh = lax.dot_general(h, whh_ref[...], (((1,), (1,)), ((), ())),
                         preferred_element_type=jnp.float32) + bhh_ref[...]
    ir, iz, inn = gi[:, 0:D], gi[:, D:2 * D], gi[:, 2 * D:3 * D]
    hr, hz, hn = gh[:, 0:D], gh[:, D:2 * D], gh[:, 2 * D:3 * D]
    r = jax.nn.sigmoid(ir + hr)
    z = jax.nn.sigmoid(iz + hz)
    ncand = jnp.tanh(inn + r * hn)
    hnew = (1.0 - z) * ncand + z * h
    msel = jax.lax.broadcasted_iota(jnp.int32, mask_ref.shape, 1) == k
    mm = jnp.sum(jnp.where(msel, mask_ref[...], 0.0), axis=1, keepdims=True)
    h_ref[...] = mm * hnew + (1.0 - mm) * h


def _gru_pallas(x_steps, mask_nt, wih, whh, bih, bhh):
    grid = (N_NODES // GRU_NT, MAX_DEG)
    return pl.pallas_call(
        _gru_body,
        grid=grid,
        in_specs=[
            pl.BlockSpec((1, GRU_NT, D), lambda i, k: (k, i, 0)),
            pl.BlockSpec((GRU_NT, MAX_DEG), lambda i, k: (i, 0)),
            pl.BlockSpec((3 * D, D), lambda i, k: (0, 0)),
            pl.BlockSpec((3 * D, D), lambda i, k: (0, 0)),
            pl.BlockSpec((1, 3 * D), lambda i, k: (0, 0)),
            pl.BlockSpec((1, 3 * D), lambda i, k: (0, 0)),
        ],
        out_specs=pl.BlockSpec((GRU_NT, D), lambda i, k: (i, 0)),
        out_shape=jax.ShapeDtypeStruct((N_NODES, D), jnp.float32),
    )(x_steps, mask_nt, wih, whh, bih[None, :], bhh[None, :])


def _gru_neigh(h, src, dst, p):
    E = src.shape[0]
    order = jnp.argsort(dst)
    dst_s = dst[order]
    src_s = src[order]
    starts = jnp.searchsorted(dst_s, jnp.arange(N_NODES))
    ends = jnp.searchsorted(dst_s, jnp.arange(N_NODES), side='right')
    pos = starts[None, :] + jnp.arange(MAX_DEG)[:, None]
    valid = pos < ends[None, :]
    idx_steps = jnp.where(valid,
                          src_s[jnp.minimum(pos, E - 1)].astype(jnp.int32), 0)
    mask_nt = valid.T.astype(jnp.float32)
    x_steps = h.astype(jnp.bfloat16)[idx_steps.reshape(-1)].reshape(
        MAX_DEG, N_NODES, D)
    return _gru_pallas(x_steps, mask_nt, p['W_ih'], p['W_hh'], p['b_ih'], p['b_hh'])


def _build_tree_consts():
    cks, cbs = [], []
    for k in range(TREE_DEPTH):
        C = np.zeros((NUM_LEAVES, NUM_LEAVES), np.float32)
        bvec = np.zeros((NUM_LEAVES,), np.float32)
        for l in range(NUM_LEAVES):
            prefix = l >> (TREE_DEPTH - k)
            n = (1 << k) + prefix
            jk = (l >> (TREE_DEPTH - 1 - k)) & 1
            C[n, l] = 1.0 if jk == 0 else -1.0
            bvec[l] = 0.0 if jk == 0 else 1.0
        cks.append(np.kron(np.eye(NUM_TREES, dtype=np.float32), C))
        cbs.append(np.tile(bvec, NUM_TREES))
    return np.stack(cks), np.stack(cbs)


_C_BD_NP, _C_BIAS_NP = _build_tree_consts()


def _mu_body(sr_ref, wde_ref, cbd_ref, cb_ref, srow_ref, w_ref):
    d = jax.nn.sigmoid(jnp.dot(sr_ref[...], wde_ref[...],
                               preferred_element_type=jnp.float32))
    mu = jnp.dot(d, cbd_ref[0], preferred_element_type=jnp.float32) + cb_ref[0:1, :]
    for k in range(1, TREE_DEPTH):
        mu = mu * (jnp.dot(d, cbd_ref[k], preferred_element_type=jnp.float32)
                   + cb_ref[k:k + 1, :])
    w_ref[...] = mu * (0.5 / NUM_TREES) / srow_ref[...]


def _ndf_w(sr, wd_eff, s_row):
    sr_dim = sr.shape[1]
    return pl.pallas_call(
        _mu_body,
        in_specs=[
            pl.BlockSpec((N_GRAPHS, sr_dim), lambda: (0, 0)),
            pl.BlockSpec((sr_dim, PI_ROWS), lambda: (0, 0)),
            pl.BlockSpec((TREE_DEPTH, PI_ROWS, PI_ROWS), lambda: (0, 0, 0)),
            pl.BlockSpec((TREE_DEPTH, PI_ROWS), lambda: (0, 0)),
            pl.BlockSpec((1, PI_ROWS), lambda: (0, 0)),
        ],
        out_specs=pl.BlockSpec((N_GRAPHS, PI_ROWS), lambda: (0, 0)),
        out_shape=jax.ShapeDtypeStruct((N_GRAPHS, PI_ROWS), jnp.float32),
    )(sr, wd_eff, jnp.asarray(_C_BD_NP), jnp.asarray(_C_BIAS_NP), s_row)


STATS_T = 4096
OUT_T = 2048


def _stats_body(pi_ref, m_ref, s_ref):
    j = pl.program_id(0)
    col0 = j * STATS_T
    idx = jax.lax.broadcasted_iota(jnp.int32, pi_ref.shape, 1) + col0
    x = jnp.where(idx < NUM_ITEMS, pi_ref[...], -jnp.inf)
    tile_m = jnp.max(x, axis=1, keepdims=True)

    @pl.when(j == 0)
    def _():
        m_ref[...] = jnp.full_like(m_ref, -jnp.inf)
        s_ref[...] = jnp.zeros_like(s_ref)

    m_old = m_ref[...]
    m_new = jnp.maximum(m_old, tile_m)
    t_s = jnp.sum(jnp.exp(x - m_new), axis=1, keepdims=True)
    s_ref[...] = s_ref[...] * jnp.exp(m_old - m_new) + t_s
    m_ref[...] = m_new


def _pi_stats(pi_r):
    grid = (pl.cdiv(NUM_ITEMS, STATS_T),)
    return pl.pallas_call(
        _stats_body,
        grid=grid,
        in_specs=[pl.BlockSpec((PI_ROWS, STATS_T), lambda j: (0, j))],
        out_specs=[
            pl.BlockSpec((PI_ROWS, 1), lambda j: (0, 0)),
            pl.BlockSpec((PI_ROWS, 1), lambda j: (0, 0)),
        ],
        out_shape=[
            jax.ShapeDtypeStruct((PI_ROWS, 1), jnp.float32),
            jax.ShapeDtypeStruct((PI_ROWS, 1), jnp.float32),
        ],
    )(pi_r)


def _logits_body(w_ref, sr2_ref, m_ref, pi_ref, emb_ref, out_ref):
    e = emb_ref[...]
    nrm = jnp.sqrt(jnp.sum(e * e, axis=1, keepdims=True))
    scale = jnp.minimum(1.0, 1.0 / jnp.maximum(nrm, 1e-12))
    en = e * scale
    expp = jnp.exp(pi_ref[...] - m_ref[...])
    acc = jax.lax.dot_general(
        sr2_ref[...], en, (((1,), (1,)), ((), ())),
        preferred_element_type=jnp.float32)
    acc = acc + jax.lax.dot(w_ref[...], expp, preferred_element_type=jnp.float32)
    out_ref[...] = acc


def _fused_logits(w, sr2h, m, pi_r, emb):
    grid = (pl.cdiv(NUM_ITEMS, OUT_T),)
    return pl.pallas_call(
        _logits_body,
        grid=grid,
        in_specs=[
            pl.BlockSpec((PI_ROWS, PI_ROWS), lambda j: (0, 0)),
            pl.BlockSpec((N_GRAPHS, D), lambda j: (0, 0)),
            pl.BlockSpec((PI_ROWS, 1), lambda j: (0, 0)),
            pl.BlockSpec((PI_ROWS, OUT_T), lambda j: (0, j)),
            pl.BlockSpec((OUT_T, D), lambda j: (j, 0)),
        ],
        out_specs=pl.BlockSpec((N_GRAPHS, OUT_T), lambda j: (0, j)),
        out_shape=jax.ShapeDtypeStruct((N_GRAPHS, NUM_ITEMS), jnp.float32),
    )(w, sr2h, m, pi_r, emb)


def kernel(params, iid, edge_index_mg, edge_index_sg, segment_ids, last_nodes, rf_feat_idx):
    p = params
    emb = p['emb']
    fe = emb[iid]
    fn = jnp.linalg.norm(fe, axis=-1, keepdims=True)
    feat = fe * jnp.minimum(1.0, 1.0 / jnp.maximum(fn, 1e-12))

    h = _bn(feat, p['bn0_g'], p['bn0_b'])
    neigh = _gru_neigh(h, edge_index_mg[0], edge_index_mg[1], p)
    out = h @ p['fc_self'].T + neigh @ p['fc_neigh'].T
    out = _prelu(out, p['prelu0'])
    feat = jnp.concatenate([out, feat], axis=1)

    h = _bn(feat, p['bn1_g'], p['bn1_b'])
    q = h @ p['Wq'].T + p['bq']
    k = h @ p['Wk'].T
    v = h @ p['Wv'].T
    src, dst = edge_index_sg[0], edge_index_sg[1]
    qv = jnp.concatenate([q, v], axis=1)
    g_src = qv[src]
    e = jax.nn.sigmoid(g_src[:, :D] + k[dst]) @ p['We_sg'].T
    M = jnp.abs(p['We_sg']).sum()
    ex = jnp.exp(e[:, 0] - M)
    num = jax.ops.segment_sum(g_src[:, D:] * ex[:, None], dst, num_segments=N_NODES)
    den = jax.ops.segment_sum(ex, dst, num_segments=N_NODES)
    out = num / jnp.maximum(den, 1e-12)[:, None]
    out = _prelu(out, p['prelu1'])
    feat = jnp.concatenate([out, feat], axis=1)

    feat = jnp.concatenate([feat, jnp.zeros((feat.shape[0], D), jnp.float32)], axis=1)

    hr = _bn(feat, p['bnr_g'], p['bnr_b'])
    fu = hr @ p['Wu'].T
    fv = (hr[last_nodes] @ p['Wv_r'].T + p['bv_r'])[segment_ids]
    er = jax.nn.sigmoid(fu + fv) @ p['We_r'].T
    Mr = jnp.abs(p['We_r']).sum()
    exr = jnp.exp(er[:, 0] - Mr)
    num_r = jax.ops.segment_sum(hr * exr[:, None], segment_ids, num_segments=N_GRAPHS)
    den_r = jax.ops.segment_sum(exr, segment_ids, num_segments=N_GRAPHS)
    rst = num_r / jnp.maximum(den_r, 1e-12)[:, None]
    sr_g = _prelu(rst @ p['Wout_r'].T, p['prelu_r'])
    sr_l = feat[last_nodes]
    sr = jnp.concatenate([sr_l, sr_g], axis=1)

    srn = _bn(sr, p['bnf_g'], p['bnf_b'])
    sr2h = 0.5 * (srn @ p['fc_sr'].T)

    pi_r = p['rf_pi'].reshape(PI_ROWS, NUM_ITEMS)
    m, s = _pi_stats(pi_r)
    sr_dim = sr.shape[1]
    wd_eff = jnp.zeros((NUM_TREES, sr_dim, NUM_LEAVES), jnp.float32).at[
        jnp.arange(NUM_TREES)[:, None], rf_feat_idx].add(p['rf_Wd'])
    wd_eff = jnp.transpose(wd_eff, (1, 0, 2)).reshape(sr_dim, PI_ROWS)
    w = _ndf_w(sr, wd_eff, s[:, 0][None, :])
    return _fused_logits(w, sr2h, m, pi_r, emb)


# speedup vs baseline: 1.2674x; 1.2674x over previous
import functools
import numpy as np
import jax
import jax.numpy as jnp
from jax import lax
from jax.experimental import pallas as pl
from jax.experimental.pallas import tpu as pltpu
from jax.experimental.pallas import tpu_sc as plsc

N_NODES = 10000
N_GRAPHS = 512
NUM_ITEMS = 100000
D = 128
NUM_TREES = 16
TREE_DEPTH = 5
NUM_LEAVES = 32
MAX_DEG = 8
EPS = 1e-5
PI_ROWS = NUM_TREES * NUM_LEAVES


def _bn(x, g, b):
    m = x.mean(axis=0)
    v = x.var(axis=0)
    return g * (x - m) / jnp.sqrt(v + EPS) + b


def _prelu(x, a):
    return jnp.where(x > 0, x, a * x)


def _seg_softmax(e, seg, n):
    mx = jax.ops.segment_max(e, seg, num_segments=n)
    mx = jnp.where(jnp.isfinite(mx), mx, 0.0)
    ex = jnp.exp(e - mx[seg])
    s = jax.ops.segment_sum(ex, seg, num_segments=n)
    return ex / jnp.maximum(s[seg], 1e-12)


NW = 32


GATHER_NB = 4


def _sc_gather_rows(table, idx2, n_rows):
    nch = n_rows // 128
    assert nch % (NW * GATHER_NB) == 0
    nc_w = nch // NW
    mesh = plsc.VectorSubcoreMesh(core_axis_name="c", subcore_axis_name="s")

    @functools.partial(
        pl.kernel, mesh=mesh,
        out_type=jax.ShapeDtypeStruct((n_rows, 128), jnp.float32),
        scratch_types=(
            [pltpu.VMEM((128,), jnp.int32) for _ in range(GATHER_NB)]
            + [pltpu.VMEM((128, 128), jnp.float32) for _ in range(GATHER_NB)]
            + [pltpu.SemaphoreType.DMA]
        ),
    )
    def gk(table_hbm, idx_hbm, out_hbm, *rest):
        idxb = rest[:GATHER_NB]
        bufs = rest[GATHER_NB:2 * GATHER_NB]
        sem = rest[2 * GATHER_NB]
        wid = lax.axis_index("s") * 2 + lax.axis_index("c")
        c0 = wid * nc_w
        for g in range(0, nc_w, GATHER_NB):
            for b in range(GATHER_NB):
                pltpu.sync_copy(
                    idx_hbm.at[pl.ds((c0 + g + b) * 128, 128)], idxb[b])
            descs = [
                pltpu.async_copy(table_hbm.at[idxb[b]], bufs[b], sem)
                for b in range(GATHER_NB)
            ]
            for b in range(GATHER_NB):
                descs[b].wait()
                pltpu.sync_copy(
                    bufs[b], out_hbm.at[pl.ds((c0 + g + b) * 128, 128)])

    return gk(table, idx2)


def _pad_idx(idx, mult):
    n = idx.shape[0]
    npad = ((n + mult - 1) // mult) * mult
    return jnp.pad(idx, (0, npad - n)), npad


GRU_NT = 2000


def _gru_body(x_ref, mask_ref, wih_ref, whh_ref, bih_ref, bhh_ref, h_ref):
    k = pl.program_id(1)

    @pl.when(k == 0)
    def _():
        h_ref[...] = jnp.zeros_like(h_ref)

    x = x_ref[0].astype(jnp.float32)
    h = h_ref[...]
    gi = lax.dot_general(x, wih_ref[...], (((1,), (1,)), ((), ())),
                         preferred_element_type=jnp.float32) + bih_ref[...]
    gh = lax.dot_general(h, whh_ref[...], (((1,), (1,)), ((), ())),
                         preferred_element_type=jnp.float32) + bhh_ref[...]
    ir, iz, inn = gi[:, 0:D], gi[:, D:2 * D], gi[:, 2 * D:3 * D]
    hr, hz, hn = gh[:, 0:D], gh[:, D:2 * D], gh[:, 2 * D:3 * D]
    r = jax.nn.sigmoid(ir + hr)
    z = jax.nn.sigmoid(iz + hz)
    ncand = jnp.tanh(inn + r * hn)
    hnew = (1.0 - z) * ncand + z * h
    msel = jax.lax.broadcasted_iota(jnp.int32, mask_ref.shape, 1) == k
    mm = jnp.sum(jnp.where(msel, mask_ref[...], 0.0), axis=1, keepdims=True)
    h_ref[...] = mm * hnew + (1.0 - mm) * h


def _gru_pallas(x_steps, mask_nt, wih, whh, bih, bhh):
    grid = (N_NODES // GRU_NT, MAX_DEG)
    return pl.pallas_call(
        _gru_body,
        grid=grid,
        in_specs=[
            pl.BlockSpec((1, GRU_NT, D), lambda i, k: (k, i, 0)),
            pl.BlockSpec((GRU_NT, MAX_DEG), lambda i, k: (i, 0)),
            pl.BlockSpec((3 * D, D), lambda i, k: (0, 0)),
            pl.BlockSpec((3 * D, D), lambda i, k: (0, 0)),
            pl.BlockSpec((1, 3 * D), lambda i, k: (0, 0)),
            pl.BlockSpec((1, 3 * D), lambda i, k: (0, 0)),
        ],
        out_specs=pl.BlockSpec((GRU_NT, D), lambda i, k: (i, 0)),
        out_shape=jax.ShapeDtypeStruct((N_NODES, D), jnp.float32),
    )(x_steps, mask_nt, wih, whh, bih[None, :], bhh[None, :])


def _gru_neigh(h, src, dst, p):
    E = src.shape[0]
    order = jnp.argsort(dst)
    dst_s = dst[order]
    src_s = src[order]
    starts = jnp.searchsorted(dst_s, jnp.arange(N_NODES))
    rank = jnp.arange(E) - starts[dst_s]
    idx_steps = jnp.zeros((MAX_DEG, N_NODES), jnp.int32).at[rank, dst_s].set(
        src_s.astype(jnp.int32))
    mask_nt = jnp.zeros((N_NODES, MAX_DEG), jnp.float32).at[dst_s, rank].set(1.0)
    x_steps = h.astype(jnp.bfloat16)[idx_steps.reshape(-1)].reshape(
        MAX_DEG, N_NODES, D)
    return _gru_pallas(x_steps, mask_nt, p['W_ih'], p['W_hh'], p['b_ih'], p['b_hh'])


def _build_tree_consts():
    cks, cbs = [], []
    for k in range(TREE_DEPTH):
        C = np.zeros((NUM_LEAVES, NUM_LEAVES), np.float32)
        bvec = np.zeros((NUM_LEAVES,), np.float32)
        for l in range(NUM_LEAVES):
            prefix = l >> (TREE_DEPTH - k)
            n = (1 << k) + prefix
            jk = (l >> (TREE_DEPTH - 1 - k)) & 1
            C[n, l] = 1.0 if jk == 0 else -1.0
            bvec[l] = 0.0 if jk == 0 else 1.0
        cks.append(np.kron(np.eye(NUM_TREES, dtype=np.float32), C))
        cbs.append(np.tile(bvec, NUM_TREES))
    return np.stack(cks), np.stack(cbs)


_C_BD_NP, _C_BIAS_NP = _build_tree_consts()


def _mu_body(sr_ref, wde_ref, cbd_ref, cb_ref, srow_ref, w_ref):
    d = jax.nn.sigmoid(jnp.dot(sr_ref[...], wde_ref[...],
                               preferred_element_type=jnp.float32))
    mu = jnp.dot(d, cbd_ref[0], preferred_element_type=jnp.float32) + cb_ref[0:1, :]
    for k in range(1, TREE_DEPTH):
        mu = mu * (jnp.dot(d, cbd_ref[k], preferred_element_type=jnp.float32)
                   + cb_ref[k:k + 1, :])
    w_ref[...] = mu * (0.5 / NUM_TREES) / srow_ref[...]


def _ndf_w(sr, wd_eff, s_row):
    sr_dim = sr.shape[1]
    return pl.pallas_call(
        _mu_body,
        in_specs=[
            pl.BlockSpec((N_GRAPHS, sr_dim), lambda: (0, 0)),
            pl.BlockSpec((sr_dim, PI_ROWS), lambda: (0, 0)),
            pl.BlockSpec((TREE_DEPTH, PI_ROWS, PI_ROWS), lambda: (0, 0, 0)),
            pl.BlockSpec((TREE_DEPTH, PI_ROWS), lambda: (0, 0)),
            pl.BlockSpec((1, PI_ROWS), lambda: (0, 0)),
        ],
        out_specs=pl.BlockSpec((N_GRAPHS, PI_ROWS), lambda: (0, 0)),
        out_shape=jax.ShapeDtypeStruct((N_GRAPHS, PI_ROWS), jnp.float32),
    )(sr, wd_eff, jnp.asarray(_C_BD_NP), jnp.asarray(_C_BIAS_NP), s_row)


STATS_T = 4096
OUT_T = 2048


def _stats_body(pi_ref, m_ref, s_ref):
    j = pl.program_id(0)
    col0 = j * STATS_T
    idx = jax.lax.broadcasted_iota(jnp.int32, pi_ref.shape, 1) + col0
    x = jnp.where(idx < NUM_ITEMS, pi_ref[...], -jnp.inf)
    tile_m = jnp.max(x, axis=1, keepdims=True)

    @pl.when(j == 0)
    def _():
        m_ref[...] = jnp.full_like(m_ref, -jnp.inf)
        s_ref[...] = jnp.zeros_like(s_ref)

    m_old = m_ref[...]
    m_new = jnp.maximum(m_old, tile_m)
    t_s = jnp.sum(jnp.exp(x - m_new), axis=1, keepdims=True)
    s_ref[...] = s_ref[...] * jnp.exp(m_old - m_new) + t_s
    m_ref[...] = m_new


def _pi_stats(pi_r):
    grid = (pl.cdiv(NUM_ITEMS, STATS_T),)
    return pl.pallas_call(
        _stats_body,
        grid=grid,
        in_specs=[pl.BlockSpec((PI_ROWS, STATS_T), lambda j: (0, j))],
        out_specs=[
            pl.BlockSpec((PI_ROWS, 1), lambda j: (0, 0)),
            pl.BlockSpec((PI_ROWS, 1), lambda j: (0, 0)),
        ],
        out_shape=[
            jax.ShapeDtypeStruct((PI_ROWS, 1), jnp.float32),
            jax.ShapeDtypeStruct((PI_ROWS, 1), jnp.float32),
        ],
    )(pi_r)


def _logits_body(w_ref, sr2_ref, m_ref, pi_ref, emb_ref, out_ref):
    e = emb_ref[...]
    nrm = jnp.sqrt(jnp.sum(e * e, axis=1, keepdims=True))
    scale = jnp.minimum(1.0, 1.0 / jnp.maximum(nrm, 1e-12))
    en = e * scale
    expp = jnp.exp(pi_ref[...] - m_ref[...])
    acc = jax.lax.dot_general(
        sr2_ref[...], en, (((1,), (1,)), ((), ())),
        preferred_element_type=jnp.float32)
    acc = acc + jax.lax.dot(w_ref[...], expp, preferred_element_type=jnp.float32)
    out_ref[...] = acc


def _fused_logits(w, sr2h, m, pi_r, emb):
    grid = (pl.cdiv(NUM_ITEMS, OUT_T),)
    return pl.pallas_call(
        _logits_body,
        grid=grid,
        in_specs=[
            pl.BlockSpec((PI_ROWS, PI_ROWS), lambda j: (0, 0)),
            pl.BlockSpec((N_GRAPHS, D), lambda j: (0, 0)),
            pl.BlockSpec((PI_ROWS, 1), lambda j: (0, 0)),
            pl.BlockSpec((PI_ROWS, OUT_T), lambda j: (0, j)),
            pl.BlockSpec((OUT_T, D), lambda j: (j, 0)),
        ],
        out_specs=pl.BlockSpec((N_GRAPHS, OUT_T), lambda j: (0, j)),
        out_shape=jax.ShapeDtypeStruct((N_GRAPHS, NUM_ITEMS), jnp.float32),
    )(w, sr2h, m, pi_r, emb)


def kernel(params, iid, edge_index_mg, edge_index_sg, segment_ids, last_nodes, rf_feat_idx):
    p = params
    emb = p['emb']
    fe = emb[iid]
    fn = jnp.linalg.norm(fe, axis=-1, keepdims=True)
    feat = fe * jnp.minimum(1.0, 1.0 / jnp.maximum(fn, 1e-12))

    h = _bn(feat, p['bn0_g'], p['bn0_b'])
    neigh = _gru_neigh(h, edge_index_mg[0], edge_index_mg[1], p)
    out = h @ p['fc_self'].T + neigh @ p['fc_neigh'].T
    out = _prelu(out, p['prelu0'])
    feat = jnp.concatenate([out, feat], axis=1)

    h = _bn(feat, p['bn1_g'], p['bn1_b'])
    q = h @ p['Wq'].T + p['bq']
    k = h @ p['Wk'].T
    v = h @ p['Wv'].T
    src, dst = edge_index_sg[0], edge_index_sg[1]
    qv = jnp.concatenate([q, v], axis=1)
    g_src = qv[src]
    e = jax.nn.sigmoid(g_src[:, :D] + k[dst]) @ p['We_sg'].T
    M = jnp.abs(p['We_sg']).sum()
    ex = jnp.exp(e[:, 0] - M)
    num = jax.ops.segment_sum(g_src[:, D:] * ex[:, None], dst, num_segments=N_NODES)
    den = jax.ops.segment_sum(ex, dst, num_segments=N_NODES)
    out = num / jnp.maximum(den, 1e-12)[:, None]
    out = _prelu(out, p['prelu1'])
    feat = jnp.concatenate([out, feat], axis=1)

    feat = jnp.concatenate([feat, jnp.zeros((feat.shape[0], D), jnp.float32)], axis=1)

    hr = _bn(feat, p['bnr_g'], p['bnr_b'])
    fu = hr @ p['Wu'].T
    fv = (hr[last_nodes] @ p['Wv_r'].T + p['bv_r'])[segment_ids]
    er = jax.nn.sigmoid(fu + fv) @ p['We_r'].T
    Mr = jnp.abs(p['We_r']).sum()
    exr = jnp.exp(er[:, 0] - Mr)
    num_r = jax.ops.segment_sum(hr * exr[:, None], segment_ids, num_segments=N_GRAPHS)
    den_r = jax.ops.segment_sum(exr, segment_ids, num_segments=N_GRAPHS)
    rst = num_r / jnp.maximum(den_r, 1e-12)[:, None]
    sr_g = _prelu(rst @ p['Wout_r'].T, p['prelu_r'])
    sr_l = feat[last_nodes]
    sr = jnp.concatenate([sr_l, sr_g], axis=1)

    srn = _bn(sr, p['bnf_g'], p['bnf_b'])
    sr2h = 0.5 * (srn @ p['fc_sr'].T)

    pi_r = p['rf_pi'].reshape(PI_ROWS, NUM_ITEMS)
    m, s = _pi_stats(pi_r)
    sr_dim = sr.shape[1]
    wd_eff = jnp.zeros((NUM_TREES, sr_dim, NUM_LEAVES), jnp.float32).at[
        jnp.arange(NUM_TREES)[:, None], rf_feat_idx].add(p['rf_Wd'])
    wd_eff = jnp.transpose(wd_eff, (1, 0, 2)).reshape(sr_dim, PI_ROWS)
    w = _ndf_w(sr, wd_eff, s[:, 0][None, :])
    return _fused_logits(w, sr2h, m, pi_r, emb)
